# Initial kernel scaffold; baseline (speedup 1.0000x reference)
import jax
import jax.numpy as jnp
from jax.experimental import pallas as pl


def kernel(nodes, sampled_neighbors, sampled_relations, node_features, weight, relation_table):
    raise NotImplementedError("write your pallas kernel here")


# speedup vs baseline: 3.2409x; 3.2409x over previous
import functools

import jax
import jax.numpy as jnp
from jax import lax
from jax.experimental import pallas as pl
from jax.experimental.pallas import tpu as pltpu
from jax.experimental.pallas import tpu_sc as plsc

NC = 2
NS = 16
NW = NC * NS
L = 16

D = 128
S = 10

CH = 64
ROWS = CH * S
GSUB = 128
NG = ROWS // GSUB


def _sc_neighbor_sum(feat_hbm, idx_hbm, out_hbm, idx_v, rows_v, acc_v, sem):
  nchunk = idx_hbm.shape[1]
  wid = lax.axis_index("s") * NC + lax.axis_index("c")
  node_base = wid * (nchunk * CH)

  def chunk_body(ci, carry):
    pltpu.sync_copy(idx_hbm.at[wid, ci], idx_v)
    copies = [
        pltpu.async_copy(feat_hbm.at[idx_v.at[k]],
                         rows_v.at[pl.ds(k * GSUB, GSUB)], sem)
        for k in range(NG)
    ]
    for c in copies:
      c.wait()

    def node_body(n, carry2):
      rbase = n * S
      for c in range(D // L):
        acc = rows_v[rbase, pl.ds(c * L, L)]
        for r in range(1, S):
          acc = acc + rows_v[rbase + r, pl.ds(c * L, L)]
        acc_v[n, pl.ds(c * L, L)] = acc * 0.1
      return carry2

    lax.fori_loop(0, CH, node_body, 0)
    pltpu.sync_copy(acc_v, out_hbm.at[pl.ds(node_base + ci * CH, CH)])
    return carry

  lax.fori_loop(0, nchunk, chunk_body, 0)


def _make_sc_kernel(nodes_pad):
  nchunk = nodes_pad // (NW * CH)
  mesh = plsc.VectorSubcoreMesh(core_axis_name="c", subcore_axis_name="s",
                                num_cores=NC, num_subcores=NS)
  return pl.kernel(
      _sc_neighbor_sum,
      out_type=jax.ShapeDtypeStruct((nodes_pad, D), jnp.float32),
      mesh=mesh,
      scratch_types=[
          pltpu.VMEM((NG, GSUB), jnp.int32),
          pltpu.VMEM((ROWS, D), jnp.float32),
          pltpu.VMEM((CH, D), jnp.float32),
          pltpu.SemaphoreType.DMA,
      ],
  )


def _tc_body(nbr_ref, rel_ref, wt_ref, table_ref, out_ref):
  bn = nbr_ref.shape[0]
  rel = rel_ref[...]
  iota = lax.broadcasted_iota(jnp.int32, (bn, D), 1)
  counts = jnp.zeros((bn, D), jnp.float32)
  for s in range(S):
    counts = counts + jnp.where(rel[:, s][:, None] == iota, 0.1, 0.0)
  out = jnp.dot(nbr_ref[...], wt_ref[...], preferred_element_type=jnp.float32)
  out = out + jnp.dot(counts, table_ref[...],
                      preferred_element_type=jnp.float32)
  out_ref[...] = jnp.maximum(out, 0.0)


def _tc_combine(nbr_sum, rel_pad, wt, table_pad, bn=256):
  nodes_pad = nbr_sum.shape[0]
  grid = (nodes_pad // bn,)
  return pl.pallas_call(
      _tc_body,
      grid=grid,
      in_specs=[
          pl.BlockSpec((bn, D), lambda i: (i, 0)),
          pl.BlockSpec((bn, S), lambda i: (i, 0)),
          pl.BlockSpec((D, D), lambda i: (0, 0)),
          pl.BlockSpec((D, D), lambda i: (0, 0)),
      ],
      out_specs=pl.BlockSpec((bn, D), lambda i: (i, 0)),
      out_shape=jax.ShapeDtypeStruct((nodes_pad, D), jnp.float32),
  )(nbr_sum, rel_pad, wt, table_pad)


def kernel(nodes, sampled_neighbors, sampled_relations, node_features, weight,
           relation_table):
  del nodes
  b, s = sampled_neighbors.shape
  assert s == S and node_features.shape[1] == D

  unit = NW * CH
  nodes_pad = ((b + unit - 1) // unit) * unit
  pad = nodes_pad - b

  idx = jnp.pad(sampled_neighbors, ((0, pad), (0, 0)))
  idx = idx.reshape(NW, nodes_pad // (NW * CH), NG, GSUB)
  nbr_sum = _make_sc_kernel(nodes_pad)(node_features, idx)

  rel_pad = jnp.pad(sampled_relations, ((0, pad), (0, 0)))
  table_pad = jnp.pad(relation_table,
                      ((0, D - relation_table.shape[0]), (0, 0)))
  out = _tc_combine(nbr_sum, rel_pad, weight.T, table_pad)
  return out[:b]


# speedup vs baseline: 3.7525x; 1.1579x over previous
import functools

import jax
import jax.numpy as jnp
from jax import lax
from jax.experimental import pallas as pl
from jax.experimental.pallas import tpu as pltpu
from jax.experimental.pallas import tpu_sc as plsc

NC = 2
NS = 16
NW = NC * NS
L = 16

D = 128
S = 10

CH = 32
ROWS = CH * S
GSUB = 64
NG = ROWS // GSUB


def _sc_neighbor_sum(feat_hbm, idx_hbm, out_hbm,
                     idx0, idx1, rows0, rows1, acc0, acc1,
                     isem0, isem1, gsem0, gsem1, osem0, osem1):
  nchunk = idx_hbm.shape[1]
  wid = lax.axis_index("s") * NC + lax.axis_index("c")
  node_base = wid * (nchunk * CH)

  def pre_idx(ci, idxv, isem):
    pltpu.async_copy(idx_hbm.at[wid, ci], idxv, isem)

  def wait_idx(idxv, isem):
    pltpu.make_async_copy(idx_hbm.at[wid, 0], idxv, isem).wait()

  def fire(idxv, rowsv, sem):
    for k in range(NG):
      pltpu.async_copy(feat_hbm.at[idxv.at[k]],
                       rowsv.at[pl.ds(k * GSUB, GSUB)], sem)

  def wait_rows(rowsv, sem):
    pltpu.make_async_copy(feat_hbm.at[pl.ds(0, ROWS)], rowsv, sem).wait()

  def wait_out(accv, sem):
    pltpu.make_async_copy(accv, out_hbm.at[pl.ds(0, CH)], sem).wait()

  def reduce_chunk(rowsv, accv):
    def node_body(n, c2):
      rbase = n * S
      for c in range(D // L):
        acc = rowsv[rbase, pl.ds(c * L, L)]
        for r in range(1, S):
          acc = acc + rowsv[rbase + r, pl.ds(c * L, L)]
        accv[n, pl.ds(c * L, L)] = acc * 0.1
      return c2
    lax.fori_loop(0, CH, node_body, 0)

  pre_idx(0, idx0, isem0)
  pre_idx(1, idx1, isem1)
  wait_idx(idx0, isem0)
  fire(idx0, rows0, gsem0)
  wait_idx(idx1, isem1)
  fire(idx1, rows1, gsem1)

  def pair_body(jj, carry):
    j = jj * 2

    wait_rows(rows0, gsem0)
    @pl.when(j + 2 < nchunk)
    def _():
      pre_idx(j + 2, idx0, isem0)
    @pl.when(jj > 0)
    def _():
      wait_out(acc0, osem0)
    reduce_chunk(rows0, acc0)
    pltpu.async_copy(acc0, out_hbm.at[pl.ds(node_base + j * CH, CH)], osem0)
    @pl.when(j + 2 < nchunk)
    def _():
      wait_idx(idx0, isem0)
      fire(idx0, rows0, gsem0)

    wait_rows(rows1, gsem1)
    @pl.when(j + 3 < nchunk)
    def _():
      pre_idx(j + 3, idx1, isem1)
    @pl.when(jj > 0)
    def _():
      wait_out(acc1, osem1)
    reduce_chunk(rows1, acc1)
    pltpu.async_copy(acc1, out_hbm.at[pl.ds(node_base + (j + 1) * CH, CH)],
                     osem1)
    @pl.when(j + 3 < nchunk)
    def _():
      wait_idx(idx1, isem1)
      fire(idx1, rows1, gsem1)
    return carry

  lax.fori_loop(0, nchunk // 2, pair_body, 0)
  wait_out(acc0, osem0)
  wait_out(acc1, osem1)


def _make_sc_kernel(nodes_pad):
  mesh = plsc.VectorSubcoreMesh(core_axis_name="c", subcore_axis_name="s",
                                num_cores=NC, num_subcores=NS)
  return pl.kernel(
      _sc_neighbor_sum,
      out_type=jax.ShapeDtypeStruct((nodes_pad, D), jnp.float32),
      mesh=mesh,
      scratch_types=[
          pltpu.VMEM((NG, GSUB), jnp.int32),
          pltpu.VMEM((NG, GSUB), jnp.int32),
          pltpu.VMEM((ROWS, D), jnp.float32),
          pltpu.VMEM((ROWS, D), jnp.float32),
          pltpu.VMEM((CH, D), jnp.float32),
          pltpu.VMEM((CH, D), jnp.float32),
          pltpu.SemaphoreType.DMA,
          pltpu.SemaphoreType.DMA,
          pltpu.SemaphoreType.DMA,
          pltpu.SemaphoreType.DMA,
          pltpu.SemaphoreType.DMA,
          pltpu.SemaphoreType.DMA,
      ],
  )


def _tc_body(nbr_ref, rel_ref, wt_ref, table_ref, out_ref):
  bn = nbr_ref.shape[0]
  rel = rel_ref[...]
  iota = lax.broadcasted_iota(jnp.int32, (bn, D), 1)
  counts = jnp.zeros((bn, D), jnp.float32)
  for s in range(S):
    counts = counts + jnp.where(rel[:, s][:, None] == iota, 0.1, 0.0)
  out = jnp.dot(nbr_ref[...], wt_ref[...], preferred_element_type=jnp.float32)
  out = out + jnp.dot(counts, table_ref[...],
                      preferred_element_type=jnp.float32)
  out_ref[...] = jnp.maximum(out, 0.0)


def _tc_combine(nbr_sum, rel_pad, wt, table_pad, bn=256):
  nodes_pad = nbr_sum.shape[0]
  grid = (nodes_pad // bn,)
  return pl.pallas_call(
      _tc_body,
      grid=grid,
      in_specs=[
          pl.BlockSpec((bn, D), lambda i: (i, 0)),
          pl.BlockSpec((bn, S), lambda i: (i, 0)),
          pl.BlockSpec((D, D), lambda i: (0, 0)),
          pl.BlockSpec((D, D), lambda i: (0, 0)),
      ],
      out_specs=pl.BlockSpec((bn, D), lambda i: (i, 0)),
      out_shape=jax.ShapeDtypeStruct((nodes_pad, D), jnp.float32),
  )(nbr_sum, rel_pad, wt, table_pad)


def kernel(nodes, sampled_neighbors, sampled_relations, node_features, weight,
           relation_table):
  del nodes
  b, s = sampled_neighbors.shape
  assert s == S and node_features.shape[1] == D

  unit = NW * CH * 2
  nodes_pad = ((b + unit - 1) // unit) * unit
  pad = nodes_pad - b

  idx = jnp.pad(sampled_neighbors, ((0, pad), (0, 0)))
  idx = idx.reshape(NW, nodes_pad // (NW * CH), NG, GSUB)
  nbr_sum = _make_sc_kernel(nodes_pad)(node_features, idx)

  rel_pad = jnp.pad(sampled_relations, ((0, pad), (0, 0)))
  table_pad = jnp.pad(relation_table,
                      ((0, D - relation_table.shape[0]), (0, 0)))
  out = _tc_combine(nbr_sum, rel_pad, weight.T, table_pad)
  return out[:b]


# speedup vs baseline: 3.8632x; 1.0295x over previous
import functools

import jax
import jax.numpy as jnp
from jax import lax
from jax.experimental import pallas as pl
from jax.experimental.pallas import tpu as pltpu
from jax.experimental.pallas import tpu_sc as plsc

NC = 2
NS = 16
NW = NC * NS
L = 16

D = 128
S = 10

CH = 32
ROWS = CH * S
GSUB = 64
NG = ROWS // GSUB

CNT0 = 84
CNT1 = 16


def _sc_neighbor_sum(feat_hbm, idx_hbm, out_hbm,
                     idx0, idx1, rows0, rows1, acc0, acc1,
                     isem0, isem1, gsem0, gsem1, osem0, osem1):
  cid = lax.axis_index("c")
  sid = lax.axis_index("s")
  cnt = jnp.where(cid == 0, CNT0, CNT1)
  start = jnp.where(cid == 0, sid * CNT0, NS * CNT0 + sid * CNT1)

  def pre_idx(ci, idxv, isem):
    pltpu.async_copy(idx_hbm.at[start + ci], idxv, isem)

  def wait_idx(idxv, isem):
    pltpu.make_async_copy(idx_hbm.at[0], idxv, isem).wait()

  def fire(idxv, rowsv, sem):
    for k in range(NG):
      pltpu.async_copy(feat_hbm.at[idxv.at[k]],
                       rowsv.at[pl.ds(k * GSUB, GSUB)], sem)

  def wait_rows(rowsv, sem):
    pltpu.make_async_copy(feat_hbm.at[pl.ds(0, ROWS)], rowsv, sem).wait()

  def wait_out(accv, sem):
    pltpu.make_async_copy(accv, out_hbm.at[pl.ds(0, CH)], sem).wait()

  def reduce_chunk(rowsv, accv):
    def node_body(n, c2):
      rbase = n * S
      for c in range(D // L):
        sl = pl.ds(c * L, L)
        v = [rowsv[rbase + r, sl] for r in range(S)]
        s01, s23 = v[0] + v[1], v[2] + v[3]
        s45, s67 = v[4] + v[5], v[6] + v[7]
        s89 = v[8] + v[9]
        accv[n, sl] = ((s01 + s23) + (s45 + s67) + s89) * 0.1
      return c2
    lax.fori_loop(0, CH, node_body, 0)

  pre_idx(0, idx0, isem0)
  pre_idx(1, idx1, isem1)
  wait_idx(idx0, isem0)
  fire(idx0, rows0, gsem0)
  wait_idx(idx1, isem1)
  fire(idx1, rows1, gsem1)

  def pair_body(jj, carry):
    j = jj * 2

    wait_rows(rows0, gsem0)
    @pl.when(j + 2 < cnt)
    def _():
      pre_idx(j + 2, idx0, isem0)
    @pl.when(jj > 0)
    def _():
      wait_out(acc0, osem0)
    reduce_chunk(rows0, acc0)
    pltpu.async_copy(acc0, out_hbm.at[pl.ds((start + j) * CH, CH)], osem0)
    @pl.when(j + 2 < cnt)
    def _():
      wait_idx(idx0, isem0)
      fire(idx0, rows0, gsem0)

    wait_rows(rows1, gsem1)
    @pl.when(j + 3 < cnt)
    def _():
      pre_idx(j + 3, idx1, isem1)
    @pl.when(jj > 0)
    def _():
      wait_out(acc1, osem1)
    reduce_chunk(rows1, acc1)
    pltpu.async_copy(acc1, out_hbm.at[pl.ds((start + j + 1) * CH, CH)],
                     osem1)
    @pl.when(j + 3 < cnt)
    def _():
      wait_idx(idx1, isem1)
      fire(idx1, rows1, gsem1)
    return carry

  lax.fori_loop(0, cnt // 2, pair_body, 0)
  wait_out(acc0, osem0)
  wait_out(acc1, osem1)


def _make_sc_kernel(nodes_pad):
  mesh = plsc.VectorSubcoreMesh(core_axis_name="c", subcore_axis_name="s",
                                num_cores=NC, num_subcores=NS)
  return pl.kernel(
      _sc_neighbor_sum,
      out_type=jax.ShapeDtypeStruct((nodes_pad, D), jnp.float32),
      mesh=mesh,
      scratch_types=[
          pltpu.VMEM((NG, GSUB), jnp.int32),
          pltpu.VMEM((NG, GSUB), jnp.int32),
          pltpu.VMEM((ROWS, D), jnp.float32),
          pltpu.VMEM((ROWS, D), jnp.float32),
          pltpu.VMEM((CH, D), jnp.float32),
          pltpu.VMEM((CH, D), jnp.float32),
          pltpu.SemaphoreType.DMA,
          pltpu.SemaphoreType.DMA,
          pltpu.SemaphoreType.DMA,
          pltpu.SemaphoreType.DMA,
          pltpu.SemaphoreType.DMA,
          pltpu.SemaphoreType.DMA,
      ],
  )


def _tc_body(nbr_ref, rel_ref, wt_ref, table_ref, out_ref):
  bn = nbr_ref.shape[0]
  rel = rel_ref[...]
  iota = lax.broadcasted_iota(jnp.int32, (bn, D), 1)
  counts = jnp.zeros((bn, D), jnp.float32)
  for s in range(S):
    counts = counts + jnp.where(rel[:, s][:, None] == iota, 0.1, 0.0)
  out = jnp.dot(nbr_ref[...], wt_ref[...], preferred_element_type=jnp.float32)
  out = out + jnp.dot(counts, table_ref[...],
                      preferred_element_type=jnp.float32)
  out_ref[...] = jnp.maximum(out, 0.0)


def _tc_combine(nbr_sum, rel_pad, wt, table_pad, bn=256):
  nodes_pad = nbr_sum.shape[0]
  grid = (nodes_pad // bn,)
  return pl.pallas_call(
      _tc_body,
      grid=grid,
      in_specs=[
          pl.BlockSpec((bn, D), lambda i: (i, 0)),
          pl.BlockSpec((bn, S), lambda i: (i, 0)),
          pl.BlockSpec((D, D), lambda i: (0, 0)),
          pl.BlockSpec((D, D), lambda i: (0, 0)),
      ],
      out_specs=pl.BlockSpec((bn, D), lambda i: (i, 0)),
      out_shape=jax.ShapeDtypeStruct((nodes_pad, D), jnp.float32),
  )(nbr_sum, rel_pad, wt, table_pad)


def kernel(nodes, sampled_neighbors, sampled_relations, node_features, weight,
           relation_table):
  del nodes
  b, s = sampled_neighbors.shape
  assert s == S and node_features.shape[1] == D

  unit = NS * (CNT0 + CNT1) * CH
  nodes_pad = ((b + unit - 1) // unit) * unit
  pad = nodes_pad - b
  total_chunks = nodes_pad // CH
  assert total_chunks == NS * (CNT0 + CNT1)

  idx = jnp.pad(sampled_neighbors, ((0, pad), (0, 0)))
  idx = idx.reshape(total_chunks, NG, GSUB)
  nbr_sum = _make_sc_kernel(nodes_pad)(node_features, idx)

  rel_pad = jnp.pad(sampled_relations, ((0, pad), (0, 0)))
  table_pad = jnp.pad(relation_table,
                      ((0, D - relation_table.shape[0]), (0, 0)))
  out = _tc_combine(nbr_sum, rel_pad, weight.T, table_pad)
  return out[:b]


# speedup vs baseline: 3.8975x; 1.0089x over previous
import functools

import jax
import jax.numpy as jnp
from jax import lax
from jax.experimental import pallas as pl
from jax.experimental.pallas import tpu as pltpu
from jax.experimental.pallas import tpu_sc as plsc

NC = 2
NS = 16
NW = NC * NS
L = 16

D = 128
S = 10

CH = 32
ROWS = CH * S
GSUB = 64
NG = ROWS // GSUB

CNT0 = 84
CNT1 = 16


def _sc_neighbor_sum(feat_hbm, idx_hbm, out_hbm,
                     idx0, idx1, rows0, rows1, acc0, acc1,
                     isem0, isem1, gsem0, gsem1, osem0, osem1):
  cid = lax.axis_index("c")
  sid = lax.axis_index("s")
  cnt = jnp.where(cid == 0, CNT0, CNT1)
  start = jnp.where(cid == 0, sid * CNT0, NS * CNT0 + sid * CNT1)

  def pre_idx(ci, idxv, isem):
    pltpu.async_copy(idx_hbm.at[start + ci], idxv, isem)

  def wait_idx(idxv, isem):
    pltpu.make_async_copy(idx_hbm.at[0], idxv, isem).wait()

  def fire(idxv, rowsv, sem):
    for k in range(NG):
      pltpu.async_copy(feat_hbm.at[idxv.at[k]],
                       rowsv.at[pl.ds(k * GSUB, GSUB)], sem)

  def wait_rows(rowsv, sem):
    pltpu.make_async_copy(feat_hbm.at[pl.ds(0, ROWS)], rowsv, sem).wait()

  def wait_out(accv, sem):
    pltpu.make_async_copy(accv, out_hbm.at[pl.ds(0, CH)], sem).wait()

  def reduce_chunk(rowsv, accv):
    def node_body(n, c2):
      rbase = n * S
      for c in range(D // L):
        sl = pl.ds(c * L, L)
        v = [rowsv[rbase + r, sl] for r in range(S)]
        s01, s23 = v[0] + v[1], v[2] + v[3]
        s45, s67 = v[4] + v[5], v[6] + v[7]
        s89 = v[8] + v[9]
        accv[n, sl] = ((s01 + s23) + (s45 + s67) + s89) * 0.1
      return c2
    lax.fori_loop(0, CH, node_body, 0)

  pre_idx(0, idx0, isem0)
  pre_idx(1, idx1, isem1)
  wait_idx(idx0, isem0)
  fire(idx0, rows0, gsem0)
  wait_idx(idx1, isem1)
  fire(idx1, rows1, gsem1)

  def pair_body(jj, carry):
    j = jj * 2

    wait_rows(rows0, gsem0)
    @pl.when(j + 2 < cnt)
    def _():
      pre_idx(j + 2, idx0, isem0)
    @pl.when(jj > 0)
    def _():
      wait_out(acc0, osem0)
    reduce_chunk(rows0, acc0)
    pltpu.async_copy(acc0, out_hbm.at[pl.ds((start + j) * CH, CH)], osem0)
    @pl.when(j + 2 < cnt)
    def _():
      wait_idx(idx0, isem0)
      fire(idx0, rows0, gsem0)

    wait_rows(rows1, gsem1)
    @pl.when(j + 3 < cnt)
    def _():
      pre_idx(j + 3, idx1, isem1)
    @pl.when(jj > 0)
    def _():
      wait_out(acc1, osem1)
    reduce_chunk(rows1, acc1)
    pltpu.async_copy(acc1, out_hbm.at[pl.ds((start + j + 1) * CH, CH)],
                     osem1)
    @pl.when(j + 3 < cnt)
    def _():
      wait_idx(idx1, isem1)
      fire(idx1, rows1, gsem1)
    return carry

  lax.fori_loop(0, cnt // 2, pair_body, 0)
  wait_out(acc0, osem0)
  wait_out(acc1, osem1)


def _make_sc_kernel(nodes_pad):
  mesh = plsc.VectorSubcoreMesh(core_axis_name="c", subcore_axis_name="s",
                                num_cores=NC, num_subcores=NS)
  return pl.kernel(
      _sc_neighbor_sum,
      out_type=jax.ShapeDtypeStruct((nodes_pad, D), jnp.float32),
      mesh=mesh,
      scratch_types=[
          pltpu.VMEM((NG, GSUB), jnp.int32),
          pltpu.VMEM((NG, GSUB), jnp.int32),
          pltpu.VMEM((ROWS, D), jnp.float32),
          pltpu.VMEM((ROWS, D), jnp.float32),
          pltpu.VMEM((CH, D), jnp.float32),
          pltpu.VMEM((CH, D), jnp.float32),
          pltpu.SemaphoreType.DMA,
          pltpu.SemaphoreType.DMA,
          pltpu.SemaphoreType.DMA,
          pltpu.SemaphoreType.DMA,
          pltpu.SemaphoreType.DMA,
          pltpu.SemaphoreType.DMA,
      ],
  )


def _tc_rel_body(rel_ref, table_ref, out_ref):
  bn = rel_ref.shape[0]
  rel = rel_ref[...]
  iota = lax.broadcasted_iota(jnp.int32, (bn, D), 1)
  counts = jnp.zeros((bn, D), jnp.float32)
  for s in range(S):
    counts = counts + jnp.where(rel[:, s][:, None] == iota, 0.1, 0.0)
  out_ref[...] = jnp.dot(counts, table_ref[...],
                         preferred_element_type=jnp.float32)


def _tc_rel(rel_pad, table_pad, bn=256):
  nodes_pad = rel_pad.shape[0]
  return pl.pallas_call(
      _tc_rel_body,
      grid=(nodes_pad // bn,),
      in_specs=[
          pl.BlockSpec((bn, S), lambda i: (i, 0)),
          pl.BlockSpec((D, D), lambda i: (0, 0)),
      ],
      out_specs=pl.BlockSpec((bn, D), lambda i: (i, 0)),
      out_shape=jax.ShapeDtypeStruct((nodes_pad, D), jnp.float32),
  )(rel_pad, table_pad)


def _tc_final_body(nbr_ref, relout_ref, wt_ref, out_ref):
  out = jnp.dot(nbr_ref[...], wt_ref[...], preferred_element_type=jnp.float32)
  out_ref[...] = jnp.maximum(out + relout_ref[...], 0.0)


def _tc_final(nbr_sum, rel_out, wt, bn=256):
  nodes_pad = nbr_sum.shape[0]
  return pl.pallas_call(
      _tc_final_body,
      grid=(nodes_pad // bn,),
      in_specs=[
          pl.BlockSpec((bn, D), lambda i: (i, 0)),
          pl.BlockSpec((bn, D), lambda i: (i, 0)),
          pl.BlockSpec((D, D), lambda i: (0, 0)),
      ],
      out_specs=pl.BlockSpec((bn, D), lambda i: (i, 0)),
      out_shape=jax.ShapeDtypeStruct((nodes_pad, D), jnp.float32),
  )(nbr_sum, rel_out, wt)


def kernel(nodes, sampled_neighbors, sampled_relations, node_features, weight,
           relation_table):
  del nodes
  b, s = sampled_neighbors.shape
  assert s == S and node_features.shape[1] == D

  unit = NS * (CNT0 + CNT1) * CH
  nodes_pad = ((b + unit - 1) // unit) * unit
  pad = nodes_pad - b
  total_chunks = nodes_pad // CH
  assert total_chunks == NS * (CNT0 + CNT1)

  idx = jnp.pad(sampled_neighbors, ((0, pad), (0, 0)))
  idx = idx.reshape(total_chunks, NG, GSUB)
  nbr_sum = _make_sc_kernel(nodes_pad)(node_features, idx)

  rel_pad = jnp.pad(sampled_relations, ((0, pad), (0, 0)))
  table_pad = jnp.pad(relation_table,
                      ((0, D - relation_table.shape[0]), (0, 0)))
  rel_out = _tc_rel(rel_pad, table_pad)
  out = _tc_final(nbr_sum, rel_out, weight.T)
  return out[:b]


# speedup vs baseline: 11.7711x; 3.0202x over previous
import functools

import jax
import jax.numpy as jnp
from jax import lax
from jax.experimental import pallas as pl
from jax.experimental.pallas import tpu as pltpu
from jax.experimental.pallas import tpu_sc as plsc

NC = 2
NS = 16
NW = NC * NS
L = 16

D = 128
S = 10

CH = 40
ROWS = CH * S
GSUB = 100
NG = ROWS // GSUB


def _sc_neighbor_sum(cnt_hi, cnt_lo, w_hi,
                     feat_hbm, idx_hbm, out_hbm,
                     idx0, idx1, rows0, rows1, acc0, acc1,
                     isem0, isem1, gsem0, gsem1, osem0, osem1):
  cid = lax.axis_index("c")
  sid = lax.axis_index("s")
  wid = sid * NC + cid
  cnt = jnp.where(wid < w_hi, cnt_hi, cnt_lo)
  start = jnp.where(wid < w_hi, wid * cnt_hi,
                    w_hi * cnt_hi + (wid - w_hi) * cnt_lo)

  def pre_idx(ci, idxv, isem):
    pltpu.async_copy(idx_hbm.at[pl.ds((start + ci) * NG, NG)], idxv, isem)

  def wait_idx(idxv, isem):
    pltpu.make_async_copy(idx_hbm.at[pl.ds(0, NG)], idxv, isem).wait()

  def fire(idxv, rowsv, sem):
    for k in range(NG):
      pltpu.async_copy(feat_hbm.at[idxv.at[k]],
                       rowsv.at[pl.ds(k * GSUB, GSUB)], sem)

  def wait_rows(rowsv, sem):
    pltpu.make_async_copy(feat_hbm.at[pl.ds(0, ROWS)], rowsv, sem).wait()

  def wait_out(accv, sem):
    pltpu.make_async_copy(accv, out_hbm.at[pl.ds(0, CH)], sem).wait()

  def reduce_chunk(rowsv, accv):
    def node_body(n, c2):
      rbase = n * S
      for c in range(D // L):
        sl = pl.ds(c * L, L)
        v = [rowsv[rbase + r, sl] for r in range(S)]
        s01, s23 = v[0] + v[1], v[2] + v[3]
        s45, s67 = v[4] + v[5], v[6] + v[7]
        s89 = v[8] + v[9]
        accv[n, sl] = ((s01 + s23) + (s45 + s67) + s89) * 0.1
      return c2
    lax.fori_loop(0, CH, node_body, 0)

  pre_idx(0, idx0, isem0)
  pre_idx(1, idx1, isem1)
  wait_idx(idx0, isem0)
  fire(idx0, rows0, gsem0)
  wait_idx(idx1, isem1)
  fire(idx1, rows1, gsem1)

  def pair_body(jj, carry):
    j = jj * 2

    wait_rows(rows0, gsem0)
    @pl.when(j + 2 < cnt)
    def _():
      pre_idx(j + 2, idx0, isem0)
    @pl.when(jj > 0)
    def _():
      wait_out(acc0, osem0)
    reduce_chunk(rows0, acc0)
    pltpu.async_copy(acc0, out_hbm.at[pl.ds((start + j) * CH, CH)], osem0)
    @pl.when(j + 2 < cnt)
    def _():
      wait_idx(idx0, isem0)
      fire(idx0, rows0, gsem0)

    wait_rows(rows1, gsem1)
    @pl.when(j + 3 < cnt)
    def _():
      pre_idx(j + 3, idx1, isem1)
    @pl.when(jj > 0)
    def _():
      wait_out(acc1, osem1)
    reduce_chunk(rows1, acc1)
    pltpu.async_copy(acc1, out_hbm.at[pl.ds((start + j + 1) * CH, CH)],
                     osem1)
    @pl.when(j + 3 < cnt)
    def _():
      wait_idx(idx1, isem1)
      fire(idx1, rows1, gsem1)
    return carry

  lax.fori_loop(0, cnt // 2, pair_body, 0)
  wait_out(acc0, osem0)
  wait_out(acc1, osem1)


def _make_sc_kernel(b, cnt_hi, cnt_lo, w_hi):
  mesh = plsc.VectorSubcoreMesh(core_axis_name="c", subcore_axis_name="s",
                                num_cores=NC, num_subcores=NS)
  return pl.kernel(
      functools.partial(_sc_neighbor_sum, cnt_hi, cnt_lo, w_hi),
      out_type=jax.ShapeDtypeStruct((b, D), jnp.float32),
      mesh=mesh,
      scratch_types=[
          pltpu.VMEM((NG, GSUB), jnp.int32),
          pltpu.VMEM((NG, GSUB), jnp.int32),
          pltpu.VMEM((ROWS, D), jnp.float32),
          pltpu.VMEM((ROWS, D), jnp.float32),
          pltpu.VMEM((CH, D), jnp.float32),
          pltpu.VMEM((CH, D), jnp.float32),
          pltpu.SemaphoreType.DMA,
          pltpu.SemaphoreType.DMA,
          pltpu.SemaphoreType.DMA,
          pltpu.SemaphoreType.DMA,
          pltpu.SemaphoreType.DMA,
          pltpu.SemaphoreType.DMA,
      ],
  )


def _tc_rel_body(rel_ref, table_ref, out_ref):
  bn = rel_ref.shape[0]
  rel = rel_ref[...]
  iota = lax.broadcasted_iota(jnp.int32, (bn, D), 1)
  counts = jnp.zeros((bn, D), jnp.float32)
  for s in range(S):
    counts = counts + jnp.where(rel[:, s][:, None] == iota, 0.1, 0.0)
  out_ref[...] = jnp.dot(counts, table_ref[...],
                         preferred_element_type=jnp.float32)


def _tc_rel(rel, table_pad, bn=400):
  b = rel.shape[0]
  return pl.pallas_call(
      _tc_rel_body,
      grid=(b // bn,),
      in_specs=[
          pl.BlockSpec((bn, S), lambda i: (i, 0)),
          pl.BlockSpec((D, D), lambda i: (0, 0)),
      ],
      out_specs=pl.BlockSpec((bn, D), lambda i: (i, 0)),
      out_shape=jax.ShapeDtypeStruct((b, D), jnp.float32),
  )(rel, table_pad)


def _tc_final_body(nbr_ref, relout_ref, w_ref, out_ref):
  out = lax.dot_general(nbr_ref[...], w_ref[...], (((1,), (1,)), ((), ())),
                        preferred_element_type=jnp.float32)
  out_ref[...] = jnp.maximum(out + relout_ref[...], 0.0)


def _tc_final(nbr_sum, rel_out, weight, bn=400):
  b = nbr_sum.shape[0]
  return pl.pallas_call(
      _tc_final_body,
      grid=(b // bn,),
      in_specs=[
          pl.BlockSpec((bn, D), lambda i: (i, 0)),
          pl.BlockSpec((bn, D), lambda i: (i, 0)),
          pl.BlockSpec((D, D), lambda i: (0, 0)),
      ],
      out_specs=pl.BlockSpec((bn, D), lambda i: (i, 0)),
      out_shape=jax.ShapeDtypeStruct((b, D), jnp.float32),
  )(nbr_sum, rel_out, weight)


def kernel(nodes, sampled_neighbors, sampled_relations, node_features, weight,
           relation_table):
  del nodes
  b, s = sampled_neighbors.shape
  assert s == S and node_features.shape[1] == D
  assert (b * S) % (NG * GSUB) == 0 and b % 400 == 0

  total_chunks = (b * S) // (NG * GSUB)
  assert total_chunks % 2 == 0
  npairs = total_chunks // 2
  base = npairs // NW
  w_hi = npairs - base * NW
  cnt_hi, cnt_lo = 2 * (base + 1), 2 * base
  if w_hi == 0:
    w_hi, cnt_hi = NW, cnt_lo

  idx = sampled_neighbors.reshape((b * S) // GSUB, GSUB)
  nbr_sum = _make_sc_kernel(b, cnt_hi, cnt_lo, w_hi)(node_features, idx)

  table_pad = jnp.pad(relation_table,
                      ((0, D - relation_table.shape[0]), (0, 0)))
  rel_out = _tc_rel(sampled_relations, table_pad)
  return _tc_final(nbr_sum, rel_out, weight)


# speedup vs baseline: 13.4004x; 1.1384x over previous
import functools

import jax
import jax.numpy as jnp
from jax import lax
from jax.experimental import pallas as pl
from jax.experimental.pallas import tpu as pltpu
from jax.experimental.pallas import tpu_sc as plsc

NC = 2
NS = 16
NW = NC * NS
L = 16

D = 128
S = 10

CH = 40
ROWS = CH * S
GSUB = 100
NG = ROWS // GSUB


def _sc_neighbor_sum(cnt_hi, cnt_lo, w_hi,
                     feat_hbm, idx_hbm, out_hbm,
                     idx0, idx1, rows0, rows1, acc0, acc1,
                     isem0, isem1, gsem0, gsem1, osem0, osem1):
  cid = lax.axis_index("c")
  sid = lax.axis_index("s")
  wid = sid * NC + cid
  cnt = jnp.where(wid < w_hi, cnt_hi, cnt_lo)
  start = jnp.where(wid < w_hi, wid * cnt_hi,
                    w_hi * cnt_hi + (wid - w_hi) * cnt_lo)

  def pre_idx(ci, idxv, isem):
    pltpu.async_copy(idx_hbm.at[pl.ds((start + ci) * NG, NG)], idxv, isem)

  def wait_idx(idxv, isem):
    pltpu.make_async_copy(idx_hbm.at[pl.ds(0, NG)], idxv, isem).wait()

  def fire(idxv, rowsv, sem):
    for k in range(NG):
      pltpu.async_copy(feat_hbm.at[idxv.at[k]],
                       rowsv.at[pl.ds(k * GSUB, GSUB)], sem)

  def wait_rows(rowsv, sem):
    pltpu.make_async_copy(feat_hbm.at[pl.ds(0, ROWS)], rowsv, sem).wait()

  def wait_out(accv, sem):
    pltpu.make_async_copy(accv, out_hbm.at[pl.ds(0, CH)], sem).wait()

  def reduce_chunk(rowsv, accv):
    def node_body(n, c2):
      rbase = n * S
      for c in range(D // L):
        sl = pl.ds(c * L, L)
        v = [rowsv[rbase + r, sl] for r in range(S)]
        s01, s23 = v[0] + v[1], v[2] + v[3]
        s45, s67 = v[4] + v[5], v[6] + v[7]
        s89 = v[8] + v[9]
        accv[n, sl] = ((s01 + s23) + (s45 + s67) + s89) * 0.1
      return c2
    lax.fori_loop(0, CH, node_body, 0)

  pre_idx(0, idx0, isem0)
  pre_idx(1, idx1, isem1)
  wait_idx(idx0, isem0)
  fire(idx0, rows0, gsem0)
  wait_idx(idx1, isem1)
  fire(idx1, rows1, gsem1)

  def pair_body(jj, carry):
    j = jj * 2

    wait_rows(rows0, gsem0)
    @pl.when(j + 2 < cnt)
    def _():
      pre_idx(j + 2, idx0, isem0)
    @pl.when(jj > 0)
    def _():
      wait_out(acc0, osem0)
    reduce_chunk(rows0, acc0)
    pltpu.async_copy(acc0, out_hbm.at[pl.ds((start + j) * CH, CH)], osem0)
    @pl.when(j + 2 < cnt)
    def _():
      wait_idx(idx0, isem0)
      fire(idx0, rows0, gsem0)

    wait_rows(rows1, gsem1)
    @pl.when(j + 3 < cnt)
    def _():
      pre_idx(j + 3, idx1, isem1)
    @pl.when(jj > 0)
    def _():
      wait_out(acc1, osem1)
    reduce_chunk(rows1, acc1)
    pltpu.async_copy(acc1, out_hbm.at[pl.ds((start + j + 1) * CH, CH)],
                     osem1)
    @pl.when(j + 3 < cnt)
    def _():
      wait_idx(idx1, isem1)
      fire(idx1, rows1, gsem1)
    return carry

  lax.fori_loop(0, cnt // 2, pair_body, 0)
  wait_out(acc0, osem0)
  wait_out(acc1, osem1)


def _make_sc_kernel(b, cnt_hi, cnt_lo, w_hi):
  mesh = plsc.VectorSubcoreMesh(core_axis_name="c", subcore_axis_name="s",
                                num_cores=NC, num_subcores=NS)
  return pl.kernel(
      functools.partial(_sc_neighbor_sum, cnt_hi, cnt_lo, w_hi),
      out_type=jax.ShapeDtypeStruct((b, D), jnp.float32),
      mesh=mesh,
      scratch_types=[
          pltpu.VMEM((NG, GSUB), jnp.int32),
          pltpu.VMEM((NG, GSUB), jnp.int32),
          pltpu.VMEM((ROWS, D), jnp.float32),
          pltpu.VMEM((ROWS, D), jnp.float32),
          pltpu.VMEM((CH, D), jnp.float32),
          pltpu.VMEM((CH, D), jnp.float32),
          pltpu.SemaphoreType.DMA,
          pltpu.SemaphoreType.DMA,
          pltpu.SemaphoreType.DMA,
          pltpu.SemaphoreType.DMA,
          pltpu.SemaphoreType.DMA,
          pltpu.SemaphoreType.DMA,
      ],
  )


def _tc_rel_body(rel_ref, table_ref, out_ref):
  bn = rel_ref.shape[0]
  rel = rel_ref[...]
  iota = lax.broadcasted_iota(jnp.int32, (bn, D), 1)
  counts = jnp.zeros((bn, D), jnp.float32)
  for s in range(S):
    counts = counts + jnp.where(rel[:, s][:, None] == iota, 0.1, 0.0)
  out_ref[...] = jnp.dot(counts, table_ref[...],
                         preferred_element_type=jnp.float32)


def _tc_rel(rel, table_pad, bn=1000):
  b = rel.shape[0]
  return pl.pallas_call(
      _tc_rel_body,
      grid=(b // bn,),
      in_specs=[
          pl.BlockSpec((bn, S), lambda i: (i, 0)),
          pl.BlockSpec((D, D), lambda i: (0, 0)),
      ],
      out_specs=pl.BlockSpec((bn, D), lambda i: (i, 0)),
      out_shape=jax.ShapeDtypeStruct((b, D), jnp.float32),
  )(rel, table_pad)


def _tc_final_body(nbr_ref, relout_ref, w_ref, out_ref):
  out = lax.dot_general(nbr_ref[...], w_ref[...], (((1,), (1,)), ((), ())),
                        preferred_element_type=jnp.float32)
  out_ref[...] = jnp.maximum(out + relout_ref[...], 0.0)


def _tc_final(nbr_sum, rel_out, weight, bn=1000):
  b = nbr_sum.shape[0]
  return pl.pallas_call(
      _tc_final_body,
      grid=(b // bn,),
      in_specs=[
          pl.BlockSpec((bn, D), lambda i: (i, 0)),
          pl.BlockSpec((bn, D), lambda i: (i, 0)),
          pl.BlockSpec((D, D), lambda i: (0, 0)),
      ],
      out_specs=pl.BlockSpec((bn, D), lambda i: (i, 0)),
      out_shape=jax.ShapeDtypeStruct((b, D), jnp.float32),
  )(nbr_sum, rel_out, weight)


def kernel(nodes, sampled_neighbors, sampled_relations, node_features, weight,
           relation_table):
  del nodes
  b, s = sampled_neighbors.shape
  assert s == S and node_features.shape[1] == D
  assert (b * S) % (NG * GSUB) == 0 and b % 1000 == 0

  total_chunks = (b * S) // (NG * GSUB)
  assert total_chunks % 2 == 0
  npairs = total_chunks // 2
  base = npairs // NW
  w_hi = npairs - base * NW
  cnt_hi, cnt_lo = 2 * (base + 1), 2 * base
  if w_hi == 0:
    w_hi, cnt_hi = NW, cnt_lo

  idx = sampled_neighbors.reshape((b * S) // GSUB, GSUB)
  nbr_sum = _make_sc_kernel(b, cnt_hi, cnt_lo, w_hi)(node_features, idx)

  table_pad = jnp.pad(relation_table,
                      ((0, D - relation_table.shape[0]), (0, 0)))
  rel_out = _tc_rel(sampled_relations, table_pad)
  return _tc_final(nbr_sum, rel_out, weight)
